# baseline (speedup 1.0000x reference)
import functools

import jax
import jax.numpy as jnp
from jax import lax
from jax.experimental import pallas as pl
from jax.experimental.pallas import tpu as pltpu
from jax.experimental.pallas import tpu_sc as plsc

N_FRAMES = 10000
N_LAYERS = 4
B, M, H, W = 16, 4, 128, 224
PAIRS = B * M
NC, NS, L = 2, 16, 16
NW = NC * NS
PAIRS_PER_W = PAIRS // NW
NCHUNK = 4
WC = W // NCHUNK
HG = H // L
OUT_PAIR = W * 2 * H
NSLOT = 4

_MESH = plsc.VectorSubcoreMesh(core_axis_name="c", subcore_axis_name="s")
_OUT_TYPE = jax.ShapeDtypeStruct((PAIRS * OUT_PAIR,), jnp.float32)
_NSTEP = PAIRS_PER_W * NCHUNK
_SCRATCH = [
    pltpu.VMEM((B,), jnp.int32),
    pltpu.VMEM((2, 8, 128), jnp.float32),
    pltpu.VMEM((NSLOT, WC * H), jnp.float32),
    pltpu.VMEM((NSLOT, WC * H), jnp.float32),
    pltpu.VMEM((NSLOT, WC * 2 * H), jnp.float32),
    pltpu.SemaphoreType.DMA((NSLOT,)),
    pltpu.SemaphoreType.DMA((NSLOT,)),
    pltpu.SemaphoreType.DMA,
]


def _body(theta_hbm, idx_hbm, grid_hbm, out_hbm, idx_v, th_v, xbuf, ybuf,
          gout, isem, osem, ssem):
    wid = lax.axis_index("s") * NC + lax.axis_index("c")
    b = wid // 2
    m0 = (wid % 2) * 2
    lanes = lax.iota(jnp.int32, L)
    one = jnp.full((L,), 1.0, jnp.float32)

    pltpu.sync_copy(idx_hbm, idx_v)
    idxv = idx_v[...]
    f = jnp.max(jnp.where(lanes == b, idxv, 0))
    fb = f // 128
    fr = f % 128
    fr_hi = (fr // L) * L
    fr_lo = fr % L

    for p in range(PAIRS_PER_W):
        pltpu.sync_copy(
            theta_hbm.at[m0 + p, :, pl.ds(fb * 128, 128)], th_v.at[p])

    coefs = []
    for p in range(PAIRS_PER_W):
        def coef(c, p=p, fr_hi=fr_hi, fr_lo=fr_lo):
            v = th_v[p, c, pl.ds(fr_hi, L)]
            s = jnp.max(jnp.where(lanes == fr_lo, v, -3.0e38))
            return jnp.full((L,), s, jnp.float32)

        a_ = coef(0)
        b_ = coef(1)
        tx = coef(2)
        ty = coef(3)
        k_ = coef(4) + 1e-6
        w_ = coef(5)
        vx = coef(6)
        vy = coef(7)
        ik = one / k_
        m00 = a_ * k_ + tx * vx
        m01 = a_ * w_ + b_ * ik + tx * vy
        m10 = ty * vx - b_ * k_
        m11 = a_ * ik - b_ * w_ + ty * vy
        coefs.append((m00, m01, tx, m10, m11, ty, vx, vy))

    def in_slice(s, c):
        p = s // NCHUNK
        cch = s % NCHUNK
        m = m0 + p
        w0 = cch * WC
        return grid_hbm.at[b, m, c, pl.ds(w0 * H, WC * H)]

    def start_in(s):
        slot = s % NSLOT
        for c in range(2):
            buf = xbuf if c == 0 else ybuf
            pltpu.async_copy(in_slice(s, c), buf.at[slot],
                             isem.at[slot])

    def wait_in(s):
        slot = s % NSLOT
        for c in range(2):
            buf = xbuf if c == 0 else ybuf
            pltpu.make_async_copy(in_slice(s, c), buf.at[slot],
                                  isem.at[slot]).wait()

    def out_slice(s):
        p = s // NCHUNK
        cch = s % NCHUNK
        out_base = (b * M + m0 + p) * OUT_PAIR + cch * WC * 2 * H
        return out_hbm.at[pl.ds(out_base, WC * 2 * H)]

    start_in(0)
    start_in(1)

    def step(s, carry):
        slot = s % NSLOT
        psel = jnp.full((L,), s // NCHUNK, jnp.int32) == 0
        m00, m01, tx, m10, m11, ty, vx, vy = (
            jnp.where(psel, c0, c1) for c0, c1 in zip(coefs[0], coefs[1]))

        @pl.when(s + 2 < _NSTEP)
        def _():
            start_in(s + 2)

        wait_in(s)

        @pl.when(s >= NSLOT)
        def _():
            pltpu.make_async_copy(gout.at[slot], out_slice(s - NSLOT),
                                  osem.at[slot]).wait()

        @plsc.parallel_loop(0, WC, unroll=4)
        def row_body(wi):
            ob = wi * (2 * H)
            ib = wi * H
            for g in range(HG):
                x = xbuf[slot, pl.ds(ib + g * L, L)]
                y = ybuf[slot, pl.ds(ib + g * L, L)]
                r = one / (vx * x + vy * y + one)
                gout[slot, pl.ds(ob + g * L, L)] = (
                    (m00 * x + m01 * y + tx) * r)
                gout[slot, pl.ds(ob + H + g * L, L)] = (
                    (m10 * x + m11 * y + ty) * r)

        pltpu.async_copy(gout.at[slot], out_slice(s), osem.at[slot])
        return carry

    lax.fori_loop(0, _NSTEP, step, 0)
    for s in range(_NSTEP - NSLOT, _NSTEP):
        pltpu.make_async_copy(gout.at[s % NSLOT], out_slice(s),
                              osem.at[s % NSLOT]).wait()


_planar_motion_sc = pl.kernel(
    _body, out_type=_OUT_TYPE, mesh=_MESH, scratch_types=_SCRATCH,
    compiler_params=pltpu.CompilerParams(needs_layout_passes=False))


def kernel(theta, idx, grid):
    theta_t = jnp.transpose(theta, (1, 2, 0))
    grid_t = jnp.transpose(grid, (0, 1, 4, 3, 2)).reshape(B, M, 3, W * H)
    out = _planar_motion_sc(theta_t, idx.astype(jnp.int32), grid_t)
    out5 = out.reshape(B, M, W, 2, H)
    return jnp.transpose(out5, (0, 1, 4, 2, 3))


# speedup vs baseline: 1.7810x; 1.7810x over previous
import functools

import jax
import jax.numpy as jnp
from jax import lax
from jax.experimental import pallas as pl
from jax.experimental.pallas import tpu as pltpu
from jax.experimental.pallas import tpu_sc as plsc

N_FRAMES = 10000
N_LAYERS = 4
B, M, H, W = 16, 4, 128, 224
PAIRS = B * M
NC, NS, L = 2, 16, 16
NW = NC * NS
PAIRS_PER_W = PAIRS // NW
NCHUNK = 4
WC = W // NCHUNK
HG = H // L
OUT_PAIR = W * 2 * H
NSLOT = 4

_MESH = plsc.VectorSubcoreMesh(core_axis_name="c", subcore_axis_name="s")
_OUT_TYPE = jax.ShapeDtypeStruct((PAIRS * OUT_PAIR,), jnp.float32)
_NSTEP = PAIRS_PER_W * NCHUNK
_SCRATCH = [
    pltpu.VMEM((B,), jnp.int32),
    pltpu.VMEM((2, 8, 128), jnp.float32),
    pltpu.VMEM((NSLOT, WC, H), jnp.float32),
    pltpu.VMEM((NSLOT, WC, H), jnp.float32),
    pltpu.VMEM((NSLOT, WC * 2 * H), jnp.float32),
    pltpu.SemaphoreType.DMA((NSLOT,)),
    pltpu.SemaphoreType.DMA((NSLOT,)),
    pltpu.SemaphoreType.DMA,
]


def _body(theta_hbm, idx_hbm, grid_hbm, out_hbm, idx_v, th_v, xbuf, ybuf,
          gout, isem, osem, ssem):
    wid = lax.axis_index("s") * NC + lax.axis_index("c")
    b = wid // 2
    m0 = (wid % 2) * 2
    lanes = lax.iota(jnp.int32, L)
    one = jnp.full((L,), 1.0, jnp.float32)

    pltpu.sync_copy(idx_hbm, idx_v)
    idxv = idx_v[...]
    f = jnp.max(jnp.where(lanes == b, idxv, 0))
    fb = f // 128
    fr = f % 128
    fr_hi = (fr // L) * L
    fr_lo = fr % L

    for p in range(PAIRS_PER_W):
        pltpu.sync_copy(
            theta_hbm.at[m0 + p, :, pl.ds(fb * 128, 128)], th_v.at[p])

    coefs = []
    for p in range(PAIRS_PER_W):
        def coef(c, p=p, fr_hi=fr_hi, fr_lo=fr_lo):
            v = th_v[p, c, pl.ds(fr_hi, L)]
            s = jnp.max(jnp.where(lanes == fr_lo, v, -3.0e38))
            return jnp.full((L,), s, jnp.float32)

        a_ = coef(0)
        b_ = coef(1)
        tx = coef(2)
        ty = coef(3)
        k_ = coef(4) + 1e-6
        w_ = coef(5)
        vx = coef(6)
        vy = coef(7)
        ik = one / k_
        m00 = a_ * k_ + tx * vx
        m01 = a_ * w_ + b_ * ik + tx * vy
        m10 = ty * vx - b_ * k_
        m11 = a_ * ik - b_ * w_ + ty * vy
        coefs.append((m00, m01, tx, m10, m11, ty, vx, vy))

    def in_slice(s, c):
        p = s // NCHUNK
        cch = s % NCHUNK
        m = m0 + p
        w0 = cch * WC
        return grid_hbm.at[b, m, c, pl.ds(w0, WC), :]

    def start_in(s):
        slot = s % NSLOT
        for c in range(2):
            buf = xbuf if c == 0 else ybuf
            pltpu.async_copy(in_slice(s, c), buf.at[slot],
                             isem.at[slot])

    def wait_in(s):
        slot = s % NSLOT
        for c in range(2):
            buf = xbuf if c == 0 else ybuf
            pltpu.make_async_copy(in_slice(s, c), buf.at[slot],
                                  isem.at[slot]).wait()

    def out_slice(s):
        p = s // NCHUNK
        cch = s % NCHUNK
        out_base = (b * M + m0 + p) * OUT_PAIR + cch * WC * 2 * H
        return out_hbm.at[pl.ds(out_base, WC * 2 * H)]

    start_in(0)
    start_in(1)

    def step(s, carry):
        slot = s % NSLOT
        psel = jnp.full((L,), s // NCHUNK, jnp.int32) == 0
        m00, m01, tx, m10, m11, ty, vx, vy = (
            jnp.where(psel, c0, c1) for c0, c1 in zip(coefs[0], coefs[1]))

        @pl.when(s + 2 < _NSTEP)
        def _():
            start_in(s + 2)

        wait_in(s)

        @pl.when(s >= NSLOT)
        def _():
            pltpu.make_async_copy(gout.at[slot], out_slice(s - NSLOT),
                                  osem.at[slot]).wait()

        @plsc.parallel_loop(0, WC, unroll=4)
        def row_body(wi):
            ob = wi * (2 * H)
            for g in range(HG):
                x = xbuf[slot, wi, pl.ds(g * L, L)]
                y = ybuf[slot, wi, pl.ds(g * L, L)]
                r = one / (vx * x + vy * y + one)
                gout[slot, pl.ds(ob + g * L, L)] = (
                    (m00 * x + m01 * y + tx) * r)
                gout[slot, pl.ds(ob + H + g * L, L)] = (
                    (m10 * x + m11 * y + ty) * r)

        pltpu.async_copy(gout.at[slot], out_slice(s), osem.at[slot])
        return carry

    lax.fori_loop(0, _NSTEP, step, 0)
    for s in range(_NSTEP - NSLOT, _NSTEP):
        pltpu.make_async_copy(gout.at[s % NSLOT], out_slice(s),
                              osem.at[s % NSLOT]).wait()


_planar_motion_sc = pl.kernel(
    _body, out_type=_OUT_TYPE, mesh=_MESH, scratch_types=_SCRATCH,
    compiler_params=pltpu.CompilerParams(needs_layout_passes=False))


def kernel(theta, idx, grid):
    theta_t = jnp.transpose(theta, (1, 2, 0))
    grid_t = jnp.transpose(grid, (0, 1, 4, 3, 2))
    out = _planar_motion_sc(theta_t, idx.astype(jnp.int32), grid_t)
    out5 = out.reshape(B, M, W, 2, H)
    return jnp.transpose(out5, (0, 1, 4, 2, 3))


# speedup vs baseline: 1.8594x; 1.0440x over previous
import functools

import jax
import jax.numpy as jnp
from jax import lax
from jax.experimental import pallas as pl
from jax.experimental.pallas import tpu as pltpu
from jax.experimental.pallas import tpu_sc as plsc

N_FRAMES = 10000
N_LAYERS = 4
B, M, H, W = 16, 4, 128, 224
PAIRS = B * M
NC, NS, L = 2, 16, 16
NW = NC * NS
PAIRS_PER_W = PAIRS // NW
NCHUNK = 4
WC = W // NCHUNK
HG = H // L
OUT_PAIR = W * 2 * H
NSLOT = 4

_MESH = plsc.VectorSubcoreMesh(core_axis_name="c", subcore_axis_name="s")
_OUT_TYPE = jax.ShapeDtypeStruct((PAIRS * OUT_PAIR,), jnp.float32)
_NSTEP = PAIRS_PER_W * NCHUNK
_SCRATCH = [
    pltpu.VMEM((B,), jnp.int32),
    pltpu.VMEM((2, 8, 128), jnp.float32),
    pltpu.VMEM((NSLOT, WC, H), jnp.float32),
    pltpu.VMEM((NSLOT, WC, H), jnp.float32),
    pltpu.VMEM((NSLOT, WC * 2 * H), jnp.float32),
    pltpu.SemaphoreType.DMA((NSLOT,)),
    pltpu.SemaphoreType.DMA((NSLOT,)),
    pltpu.SemaphoreType.DMA,
]


def _body(theta_hbm, idx_hbm, grid_hbm, out_hbm, idx_v, th_v, xbuf, ybuf,
          gout, isem, osem, ssem):
    wid = lax.axis_index("s") * NC + lax.axis_index("c")
    b = wid // 2
    m0 = (wid % 2) * 2
    lanes = lax.iota(jnp.int32, L)
    one = jnp.full((L,), 1.0, jnp.float32)

    def early_in(s):
        slot = s % NSLOT
        p = s // NCHUNK
        cch = s % NCHUNK
        w0 = cch * WC
        m = m0 + p
        pltpu.async_copy(grid_hbm.at[b, m, 0, pl.ds(w0, WC), :],
                         xbuf.at[slot], isem.at[slot])
        pltpu.async_copy(grid_hbm.at[b, m, 1, pl.ds(w0, WC), :],
                         ybuf.at[slot], isem.at[slot])

    early_in(0)
    early_in(1)

    pltpu.sync_copy(idx_hbm, idx_v)
    idxv = idx_v[...]
    f = jnp.max(jnp.where(lanes == b, idxv, 0))
    fb = f // 128
    fr = f % 128
    fr_hi = (fr // L) * L
    fr_lo = fr % L

    for p in range(PAIRS_PER_W):
        pltpu.sync_copy(
            theta_hbm.at[m0 + p, :, pl.ds(fb * 128, 128)], th_v.at[p])

    coefs = []
    for p in range(PAIRS_PER_W):
        def coef(c, p=p, fr_hi=fr_hi, fr_lo=fr_lo):
            v = th_v[p, c, pl.ds(fr_hi, L)]
            s = jnp.max(jnp.where(lanes == fr_lo, v, -3.0e38))
            return jnp.full((L,), s, jnp.float32)

        a_ = coef(0)
        b_ = coef(1)
        tx = coef(2)
        ty = coef(3)
        k_ = coef(4) + 1e-6
        w_ = coef(5)
        vx = coef(6)
        vy = coef(7)
        ik = one / k_
        m00 = a_ * k_ + tx * vx
        m01 = a_ * w_ + b_ * ik + tx * vy
        m10 = ty * vx - b_ * k_
        m11 = a_ * ik - b_ * w_ + ty * vy
        coefs.append((m00, m01, tx, m10, m11, ty, vx, vy))

    def in_slice(s, c):
        p = s // NCHUNK
        cch = s % NCHUNK
        m = m0 + p
        w0 = cch * WC
        return grid_hbm.at[b, m, c, pl.ds(w0, WC), :]

    def start_in(s):
        slot = s % NSLOT
        for c in range(2):
            buf = xbuf if c == 0 else ybuf
            pltpu.async_copy(in_slice(s, c), buf.at[slot],
                             isem.at[slot])

    def wait_in(s):
        slot = s % NSLOT
        for c in range(2):
            buf = xbuf if c == 0 else ybuf
            pltpu.make_async_copy(in_slice(s, c), buf.at[slot],
                                  isem.at[slot]).wait()

    def out_slice(s):
        p = s // NCHUNK
        cch = s % NCHUNK
        out_base = (b * M + m0 + p) * OUT_PAIR + cch * WC * 2 * H
        return out_hbm.at[pl.ds(out_base, WC * 2 * H)]

    def step(s, carry):
        slot = s % NSLOT
        psel = jnp.full((L,), s // NCHUNK, jnp.int32) == 0
        m00, m01, tx, m10, m11, ty, vx, vy = (
            jnp.where(psel, c0, c1) for c0, c1 in zip(coefs[0], coefs[1]))

        @pl.when(s + 2 < _NSTEP)
        def _():
            start_in(s + 2)

        wait_in(s)

        @pl.when(s >= NSLOT)
        def _():
            pltpu.make_async_copy(gout.at[slot], out_slice(s - NSLOT),
                                  osem.at[slot]).wait()

        @plsc.parallel_loop(0, WC, unroll=4)
        def row_body(wi):
            ob = wi * (2 * H)
            for g in range(HG):
                x = xbuf[slot, wi, pl.ds(g * L, L)]
                y = ybuf[slot, wi, pl.ds(g * L, L)]
                r = one / (vx * x + vy * y + one)
                gout[slot, pl.ds(ob + g * L, L)] = (
                    (m00 * x + m01 * y + tx) * r)
                gout[slot, pl.ds(ob + H + g * L, L)] = (
                    (m10 * x + m11 * y + ty) * r)

        pltpu.async_copy(gout.at[slot], out_slice(s), osem.at[slot])
        return carry

    lax.fori_loop(0, _NSTEP, step, 0)
    for s in range(_NSTEP - NSLOT, _NSTEP):
        pltpu.make_async_copy(gout.at[s % NSLOT], out_slice(s),
                              osem.at[s % NSLOT]).wait()


_planar_motion_sc = pl.kernel(
    _body, out_type=_OUT_TYPE, mesh=_MESH, scratch_types=_SCRATCH,
    compiler_params=pltpu.CompilerParams(needs_layout_passes=False))


def kernel(theta, idx, grid):
    theta_t = jnp.transpose(theta, (1, 2, 0))
    grid_t = jnp.transpose(grid, (0, 1, 4, 3, 2))
    out = _planar_motion_sc(theta_t, idx.astype(jnp.int32), grid_t)
    out5 = out.reshape(B, M, W, 2, H)
    return jnp.transpose(out5, (0, 1, 4, 2, 3))
